# Initial kernel scaffold; baseline (speedup 1.0000x reference)
import jax
import jax.numpy as jnp
from jax.experimental import pallas as pl


def kernel(g, x, W1, b1, W2, b2, Wc, bc):
    raise NotImplementedError("write your pallas kernel here")


# speedup vs baseline: 2.8422x; 2.8422x over previous
import functools
import jax
import jax.numpy as jnp
from jax import lax
from jax.experimental import pallas as pl
from jax.experimental.pallas import tpu as pltpu
from jax.experimental.pallas import tpu_sc as plsc

N = 10000
D = 128
C = 64
E = 320000

NC = 2
NS = 16
NW = NC * NS

NP = 10240
RPT = NP // NS
EB = 128
TPB = 80
EPT = TPB * EB
EP = NW * EPT
ERT = EP // NS // EB
DUMP = NP - 1

_mesh = plsc.VectorSubcoreMesh(core_axis_name="c", subcore_axis_name="s")
_f32 = jnp.float32


def _quake_rsqrt16(s):
    ii = plsc.bitcast(s, jnp.int32)
    ii = 0x5F3759DF - lax.shift_right_logical(ii, 1)
    y = plsc.bitcast(ii, _f32)
    hs = 0.5 * s
    for _ in range(3):
        y = y * (1.5 - hs * y * y)
    return jnp.where(s > 0.0, y, jnp.zeros((16,), _f32))


def _degnorm_body(gidx, norms_out, idx_v, deg_v, stage, acc_v, sh):
    cid = lax.axis_index("c")
    sid = lax.axis_index("s")

    pltpu.sync_copy(gidx.at[cid, pl.ds(sid * ERT, ERT)], idx_v)

    zeros16 = jnp.zeros((16,), _f32)

    def zb(i, carry):
        deg_v[pl.ds(i * 16, 16)] = zeros16
        return carry

    lax.fori_loop(0, NP // 16, zb, 0)

    ones16 = jnp.ones((16,), _f32)

    def sb(r, carry):
        for c in range(EB // 16):
            plsc.addupdate_scatter(deg_v, [idx_v[r, pl.ds(c * 16, 16)]],
                                   ones16)
        return carry

    lax.fori_loop(0, ERT, sb, 0)

    pltpu.sync_copy(deg_v, sh.at[sid])
    plsc.subcore_barrier()

    nb = sid * RPT
    for k in range(NS):
        pltpu.sync_copy(sh.at[k, pl.ds(nb, RPT)], stage.at[k])

    def rb(i, carry):
        s = stage[0, pl.ds(i * 16, 16)]
        for k in range(1, NS):
            s = s + stage[k, pl.ds(i * 16, 16)]
        acc_v[pl.ds(i * 16, 16)] = _quake_rsqrt16(s)
        return carry

    lax.fori_loop(0, RPT // 16, rb, 0)

    pltpu.sync_copy(acc_v, norms_out.at[cid, pl.ds(nb, RPT)])


@functools.partial(
    pl.kernel,
    out_type=jax.ShapeDtypeStruct((NC, NP), _f32),
    mesh=_mesh,
    scratch_types=[
        pltpu.VMEM((ERT, EB), jnp.int32),
        pltpu.VMEM((NP,), _f32),
        pltpu.VMEM((NS, RPT), _f32),
        pltpu.VMEM((RPT,), _f32),
        pltpu.VMEM_SHARED((NS, NP), _f32),
    ],
    compiler_params=pltpu.CompilerParams(needs_layout_passes=False, use_tc_tiling_on_sc=False),
)
def _degnorm(*args):
    _degnorm_body(*args)


def _make_agg():
    def body(gidx, m_hbm, out_hbm, srcv, dstv, rows, agg_sh, sem):
        cid = lax.axis_index("c")
        sid = lax.axis_index("s")
        wid = cid * NS + sid

        pltpu.sync_copy(gidx.at[0, pl.ds(wid * TPB, TPB)], srcv)
        pltpu.sync_copy(gidx.at[1, pl.ds(wid * TPB, TPB)], dstv)

        zeros16 = jnp.zeros((16,), _f32)

        def zb(i, carry):
            for c in range(D // 16):
                rows[i, pl.ds(c * 16, 16)] = zeros16
            return carry

        lax.fori_loop(0, EB, zb, 0)

        for t in range(RPT // EB):
            pltpu.sync_copy(rows, agg_sh.at[pl.ds(sid * RPT + t * EB, EB)])
        plsc.subcore_barrier()

        def eb(j, carry):
            pltpu.async_copy(m_hbm.at[srcv.at[j]], rows, sem).wait()
            pltpu.sync_copy(rows, agg_sh.at[dstv.at[j]], add=True)
            return carry

        lax.fori_loop(0, TPB, eb, 0)
        plsc.subcore_barrier()

        pltpu.sync_copy(agg_sh.at[pl.ds(sid * RPT, RPT)],
                        out_hbm.at[cid, pl.ds(sid * RPT, RPT)])

    return pl.kernel(
        body,
        out_type=jax.ShapeDtypeStruct((NC, NP, D), _f32),
        mesh=_mesh,
        scratch_types=[
            pltpu.VMEM((TPB, EB), jnp.int32),
            pltpu.VMEM((TPB, EB), jnp.int32),
            pltpu.VMEM((EB, D), _f32),
            pltpu.VMEM_SHARED((NP, D), _f32),
            pltpu.SemaphoreType.DMA,
        ],
        compiler_params=pltpu.CompilerParams(needs_layout_passes=False, use_tc_tiling_on_sc=False),
    )


_agg = _make_agg()


BN = 1024


def _scale_tc(x_ref, ns_ref, o_ref):
    o_ref[...] = x_ref[...] * ns_ref[...]


def _mm1_tc(agg_ref, ns_ref, nd_ref, w_ref, b_ref, o_ref):
    a = agg_ref[0] + agg_ref[1]
    p = jnp.dot(a, w_ref[...], preferred_element_type=_f32)
    o_ref[...] = jnp.maximum(nd_ref[...] * p + b_ref[...], 0.0) * ns_ref[...]


def _mm2_tc(agg_ref, nd_ref, w_ref, b_ref, wc_ref, bc_ref, h_ref, l_ref):
    a = agg_ref[0] + agg_ref[1]
    h = nd_ref[...] * jnp.dot(a, w_ref[...], preferred_element_type=_f32) \
        + b_ref[...]
    h_ref[...] = h
    l_ref[...] = jnp.dot(jnp.maximum(h, 0.0), wc_ref[...],
                         preferred_element_type=_f32) + bc_ref[...]


def _scale_call(xp, ns_col):
    return pl.pallas_call(
        _scale_tc,
        grid=(NP // BN,),
        in_specs=[
            pl.BlockSpec((BN, D), lambda i: (i, 0)),
            pl.BlockSpec((BN, 1), lambda i: (i, 0)),
        ],
        out_specs=pl.BlockSpec((BN, D), lambda i: (i, 0)),
        out_shape=jax.ShapeDtypeStruct((NP, D), _f32),
    )(xp, ns_col)


def _mm1_call(aggp, ns_col, nd_col, W, b_row):
    return pl.pallas_call(
        _mm1_tc,
        grid=(NP // BN,),
        in_specs=[
            pl.BlockSpec((NC, BN, D), lambda i: (0, i, 0)),
            pl.BlockSpec((BN, 1), lambda i: (i, 0)),
            pl.BlockSpec((BN, 1), lambda i: (i, 0)),
            pl.BlockSpec((D, D), lambda i: (0, 0)),
            pl.BlockSpec((1, D), lambda i: (0, 0)),
        ],
        out_specs=pl.BlockSpec((BN, D), lambda i: (i, 0)),
        out_shape=jax.ShapeDtypeStruct((NP, D), _f32),
    )(aggp, ns_col, nd_col, W, b_row)


def _mm2_call(aggp, nd_col, W, b_row, Wc, bc_row):
    return pl.pallas_call(
        _mm2_tc,
        grid=(NP // BN,),
        in_specs=[
            pl.BlockSpec((NC, BN, D), lambda i: (0, i, 0)),
            pl.BlockSpec((BN, 1), lambda i: (i, 0)),
            pl.BlockSpec((D, D), lambda i: (0, 0)),
            pl.BlockSpec((1, D), lambda i: (0, 0)),
            pl.BlockSpec((D, C), lambda i: (0, 0)),
            pl.BlockSpec((1, C), lambda i: (0, 0)),
        ],
        out_specs=[
            pl.BlockSpec((BN, D), lambda i: (i, 0)),
            pl.BlockSpec((BN, C), lambda i: (i, 0)),
        ],
        out_shape=[
            jax.ShapeDtypeStruct((NP, D), _f32),
            jax.ShapeDtypeStruct((NP, C), _f32),
        ],
    )(aggp, nd_col, W, b_row, Wc, bc_row)


@jax.jit
def kernel(g, x, W1, b1, W2, b2, Wc, bc):
    pad = jnp.full((2, EP - E), DUMP, jnp.int32)
    gidx = jnp.concatenate([g, pad], axis=1).reshape(2, NW * TPB, EB)
    xp = jnp.pad(x, ((0, NP - N), (0, 0)))

    norms = _degnorm(gidx)
    ns_col = norms[0].reshape(NP, 1)
    nd_col = norms[1].reshape(NP, 1)

    m1 = _scale_call(xp, ns_col)
    aggp1 = _agg(gidx, m1)
    m2 = _mm1_call(aggp1, ns_col, nd_col, W1, b1.reshape(1, D))
    aggp2 = _agg(gidx, m2)
    h, logits = _mm2_call(aggp2, nd_col, W2, b2.reshape(1, D),
                          Wc, bc.reshape(1, C))
    return h[:N], logits[:N]


# speedup vs baseline: 8.2297x; 2.8956x over previous
import functools
import jax
import jax.numpy as jnp
from jax import lax
from jax.experimental import pallas as pl
from jax.experimental.pallas import tpu as pltpu
from jax.experimental.pallas import tpu_sc as plsc

N = 10000
D = 128
C = 64
E = 320000

NC = 2
NS = 16
NW = NC * NS

NP = 10240
RPT = NP // NS
EB = 128
TPB = 80
EPT = TPB * EB
EP = NW * EPT
ERT = EP // NS // EB
DUMP = NP - 1

_mesh = plsc.VectorSubcoreMesh(core_axis_name="c", subcore_axis_name="s")
_f32 = jnp.float32


def _quake_rsqrt16(s):
    ii = plsc.bitcast(s, jnp.int32)
    ii = 0x5F3759DF - lax.shift_right_logical(ii, 1)
    y = plsc.bitcast(ii, _f32)
    hs = 0.5 * s
    for _ in range(3):
        y = y * (1.5 - hs * y * y)
    return jnp.where(s > 0.0, y, jnp.zeros((16,), _f32))


def _degnorm_body(gidx, norms_out, idx_v, deg_v, stage, acc_v, sh):
    cid = lax.axis_index("c")
    sid = lax.axis_index("s")

    pltpu.sync_copy(gidx.at[cid, pl.ds(sid * ERT, ERT)], idx_v)

    zeros16 = jnp.zeros((16,), _f32)

    def zb(i, carry):
        deg_v[pl.ds(i * 16, 16)] = zeros16
        return carry

    lax.fori_loop(0, NP // 16, zb, 0)

    ones16 = jnp.ones((16,), _f32)

    def sb(r, carry):
        for c in range(EB // 16):
            plsc.addupdate_scatter(deg_v, [idx_v[r, pl.ds(c * 16, 16)]],
                                   ones16)
        return carry

    lax.fori_loop(0, ERT, sb, 0)

    pltpu.sync_copy(deg_v, sh.at[sid])
    plsc.subcore_barrier()

    nb = sid * RPT
    for k in range(NS):
        pltpu.sync_copy(sh.at[k, pl.ds(nb, RPT)], stage.at[k])

    def rb(i, carry):
        s = stage[0, pl.ds(i * 16, 16)]
        for k in range(1, NS):
            s = s + stage[k, pl.ds(i * 16, 16)]
        acc_v[pl.ds(i * 16, 16)] = _quake_rsqrt16(s)
        return carry

    lax.fori_loop(0, RPT // 16, rb, 0)

    pltpu.sync_copy(acc_v, norms_out.at[cid, pl.ds(nb, RPT)])


@functools.partial(
    pl.kernel,
    out_type=jax.ShapeDtypeStruct((NC, NP), _f32),
    mesh=_mesh,
    scratch_types=[
        pltpu.VMEM((ERT, EB), jnp.int32),
        pltpu.VMEM((NP,), _f32),
        pltpu.VMEM((NS, RPT), _f32),
        pltpu.VMEM((RPT,), _f32),
        pltpu.VMEM_SHARED((NS, NP), _f32),
    ],
    compiler_params=pltpu.CompilerParams(needs_layout_passes=False, use_tc_tiling_on_sc=False),
)
def _degnorm(*args):
    _degnorm_body(*args)


def _make_agg():
    def body(gidx, m_hbm, out_hbm, srcv, dstv, rows, agg_sh, sem):
        cid = lax.axis_index("c")
        sid = lax.axis_index("s")
        wid = cid * NS + sid

        pltpu.sync_copy(gidx.at[0, pl.ds(wid * TPB, TPB)], srcv)
        pltpu.sync_copy(gidx.at[1, pl.ds(wid * TPB, TPB)], dstv)

        zeros16 = jnp.zeros((16,), _f32)

        def zb(i, carry):
            for c in range(D // 16):
                rows[i, pl.ds(c * 16, 16)] = zeros16
            return carry

        lax.fori_loop(0, EB, zb, 0)

        for t in range(RPT // EB):
            pltpu.sync_copy(rows, agg_sh.at[pl.ds(sid * RPT + t * EB, EB)])
        plsc.subcore_barrier()

        def eb(j, carry):
            pltpu.async_copy(m_hbm.at[srcv.at[j]], rows, sem).wait()
            pltpu.sync_copy(rows, agg_sh.at[dstv.at[j]], add=True)
            return carry

        lax.fori_loop(0, TPB, eb, 0)
        plsc.subcore_barrier()

        pltpu.sync_copy(agg_sh.at[pl.ds(sid * RPT, RPT)],
                        out_hbm.at[cid, pl.ds(sid * RPT, RPT)])

    return pl.kernel(
        body,
        out_type=jax.ShapeDtypeStruct((NC, NP, D), _f32),
        mesh=_mesh,
        scratch_types=[
            pltpu.VMEM((TPB, EB), jnp.int32),
            pltpu.VMEM((TPB, EB), jnp.int32),
            pltpu.VMEM((EB, D), _f32),
            pltpu.VMEM_SHARED((NP, D), _f32),
            pltpu.SemaphoreType.DMA,
        ],
        compiler_params=pltpu.CompilerParams(needs_layout_passes=False, use_tc_tiling_on_sc=False),
    )


_agg = _make_agg()


BN = 1024


def _scale_tc(x_ref, ns_ref, o_ref):
    o_ref[...] = x_ref[...] * ns_ref[...]


def _mm1_tc(agg_ref, ns_ref, nd_ref, w_ref, b_ref, o_ref):
    a = agg_ref[0] + agg_ref[1]
    p = jnp.dot(a, w_ref[...], preferred_element_type=_f32)
    o_ref[...] = jnp.maximum(nd_ref[...] * p + b_ref[...], 0.0) * ns_ref[...]


def _mm2_tc(agg_ref, nd_ref, w_ref, b_ref, wc_ref, bc_ref, h_ref, l_ref):
    a = agg_ref[0] + agg_ref[1]
    h = nd_ref[...] * jnp.dot(a, w_ref[...], preferred_element_type=_f32) \
        + b_ref[...]
    h_ref[...] = h
    l_ref[...] = jnp.dot(jnp.maximum(h, 0.0), wc_ref[...],
                         preferred_element_type=_f32) + bc_ref[...]


def _scale_call(xp, ns_col):
    return pl.pallas_call(
        _scale_tc,
        grid=(NP // BN,),
        in_specs=[
            pl.BlockSpec((BN, D), lambda i: (i, 0)),
            pl.BlockSpec((BN, 1), lambda i: (i, 0)),
        ],
        out_specs=pl.BlockSpec((BN, D), lambda i: (i, 0)),
        out_shape=jax.ShapeDtypeStruct((NP, D), _f32),
    )(xp, ns_col)


def _mm1_call(aggp, ns_col, nd_col, W, b_row):
    return pl.pallas_call(
        _mm1_tc,
        grid=(NP // BN,),
        in_specs=[
            pl.BlockSpec((NC, BN, D), lambda i: (0, i, 0)),
            pl.BlockSpec((BN, 1), lambda i: (i, 0)),
            pl.BlockSpec((BN, 1), lambda i: (i, 0)),
            pl.BlockSpec((D, D), lambda i: (0, 0)),
            pl.BlockSpec((1, D), lambda i: (0, 0)),
        ],
        out_specs=pl.BlockSpec((BN, D), lambda i: (i, 0)),
        out_shape=jax.ShapeDtypeStruct((NP, D), _f32),
    )(aggp, ns_col, nd_col, W, b_row)


def _mm2_call(aggp, nd_col, W, b_row, Wc, bc_row):
    return pl.pallas_call(
        _mm2_tc,
        grid=(NP // BN,),
        in_specs=[
            pl.BlockSpec((NC, BN, D), lambda i: (0, i, 0)),
            pl.BlockSpec((BN, 1), lambda i: (i, 0)),
            pl.BlockSpec((D, D), lambda i: (0, 0)),
            pl.BlockSpec((1, D), lambda i: (0, 0)),
            pl.BlockSpec((D, C), lambda i: (0, 0)),
            pl.BlockSpec((1, C), lambda i: (0, 0)),
        ],
        out_specs=[
            pl.BlockSpec((BN, D), lambda i: (i, 0)),
            pl.BlockSpec((BN, C), lambda i: (i, 0)),
        ],
        out_shape=[
            jax.ShapeDtypeStruct((NP, D), _f32),
            jax.ShapeDtypeStruct((NP, C), _f32),
        ],
    )(aggp, nd_col, W, b_row, Wc, bc_row)


@jax.jit
def kernel(g, x, W1, b1, W2, b2, Wc, bc):
    pad_nodes = N + jax.lax.rem(jnp.arange(EP - E, dtype=jnp.int32),
                                jnp.int32(NP - N))
    pad = jnp.stack([pad_nodes, pad_nodes])
    gidx = jnp.concatenate([g, pad], axis=1).reshape(2, NW * TPB, EB)
    xp = jnp.pad(x, ((0, NP - N), (0, 0)))

    norms = _degnorm(gidx)
    ns_col = norms[0].reshape(NP, 1)
    nd_col = norms[1].reshape(NP, 1)

    m1 = _scale_call(xp, ns_col)
    aggp1 = _agg(gidx, m1)
    m2 = _mm1_call(aggp1, ns_col, nd_col, W1, b1.reshape(1, D))
    aggp2 = _agg(gidx, m2)
    h, logits = _mm2_call(aggp2, nd_col, W2, b2.reshape(1, D),
                          Wc, bc.reshape(1, C))
    return h[:N], logits[:N]


# speedup vs baseline: 10.3703x; 1.2601x over previous
import functools
import jax
import jax.numpy as jnp
from jax import lax
from jax.experimental import pallas as pl
from jax.experimental.pallas import tpu as pltpu
from jax.experimental.pallas import tpu_sc as plsc

N = 10000
D = 128
C = 64
E = 320000

NC = 2
NS = 16
NW = NC * NS

NP = 10240
RPT = NP // NS
EB = 64
TPB = 160
EPT = TPB * EB
EP = NW * EPT
ERT = EP // NS // EB
DUMP = NP - 1

_mesh = plsc.VectorSubcoreMesh(core_axis_name="c", subcore_axis_name="s")
_f32 = jnp.float32


def _quake_rsqrt16(s):
    ii = plsc.bitcast(s, jnp.int32)
    ii = 0x5F3759DF - lax.shift_right_logical(ii, 1)
    y = plsc.bitcast(ii, _f32)
    hs = 0.5 * s
    for _ in range(3):
        y = y * (1.5 - hs * y * y)
    return jnp.where(s > 0.0, y, jnp.zeros((16,), _f32))


def _degnorm_body(gidx, norms_out, idx_v, deg_v, stage, acc_v, sh):
    cid = lax.axis_index("c")
    sid = lax.axis_index("s")

    pltpu.sync_copy(gidx.at[cid, pl.ds(sid * ERT, ERT)], idx_v)

    zeros16 = jnp.zeros((16,), _f32)

    def zb(i, carry):
        deg_v[pl.ds(i * 16, 16)] = zeros16
        return carry

    lax.fori_loop(0, NP // 16, zb, 0)

    ones16 = jnp.ones((16,), _f32)

    def sb(r, carry):
        for c in range(EB // 16):
            plsc.addupdate_scatter(deg_v, [idx_v[r, pl.ds(c * 16, 16)]],
                                   ones16)
        return carry

    lax.fori_loop(0, ERT, sb, 0)

    pltpu.sync_copy(deg_v, sh.at[sid])
    plsc.subcore_barrier()

    nb = sid * RPT
    for k in range(NS):
        pltpu.sync_copy(sh.at[k, pl.ds(nb, RPT)], stage.at[k])

    def rb(i, carry):
        s = stage[0, pl.ds(i * 16, 16)]
        for k in range(1, NS):
            s = s + stage[k, pl.ds(i * 16, 16)]
        acc_v[pl.ds(i * 16, 16)] = _quake_rsqrt16(s)
        return carry

    lax.fori_loop(0, RPT // 16, rb, 0)

    pltpu.sync_copy(acc_v, norms_out.at[cid, pl.ds(nb, RPT)])


@functools.partial(
    pl.kernel,
    out_type=jax.ShapeDtypeStruct((NC, NP), _f32),
    mesh=_mesh,
    scratch_types=[
        pltpu.VMEM((ERT, EB), jnp.int32),
        pltpu.VMEM((NP,), _f32),
        pltpu.VMEM((NS, RPT), _f32),
        pltpu.VMEM((RPT,), _f32),
        pltpu.VMEM_SHARED((NS, NP), _f32),
    ],
    compiler_params=pltpu.CompilerParams(needs_layout_passes=False, use_tc_tiling_on_sc=False),
)
def _degnorm(*args):
    _degnorm_body(*args)


def _make_agg():
    def body(gidx, m_hbm, out_hbm, srcv, dstv, rows_a, rows_b,
             agg_sh, sem_a, sem_b):
        cid = lax.axis_index("c")
        sid = lax.axis_index("s")
        wid = cid * NS + sid

        pltpu.sync_copy(gidx.at[0, pl.ds(wid * TPB, TPB)],
                        srcv.at[pl.ds(0, TPB)])
        pltpu.sync_copy(gidx.at[0, pl.ds(wid * TPB, 1)],
                        srcv.at[pl.ds(TPB, 1)])
        pltpu.sync_copy(gidx.at[1, pl.ds(wid * TPB, TPB)], dstv)

        zeros16 = jnp.zeros((16,), _f32)

        def zb(i, carry):
            for c in range(D // 16):
                rows_a[i, pl.ds(c * 16, 16)] = zeros16
            return carry

        lax.fori_loop(0, EB, zb, 0)

        for t in range(RPT // EB):
            pltpu.sync_copy(rows_a, agg_sh.at[pl.ds(sid * RPT + t * EB, EB)])
        plsc.subcore_barrier()

        pltpu.async_copy(m_hbm.at[srcv.at[0]], rows_a, sem_a)

        def eb2(t, carry):
            j = 2 * t
            pltpu.async_copy(m_hbm.at[srcv.at[j + 1]], rows_b, sem_b)
            pltpu.make_async_copy(m_hbm.at[srcv.at[j]], rows_a, sem_a).wait()
            pltpu.sync_copy(rows_a, agg_sh.at[dstv.at[j]], add=True)
            pltpu.async_copy(m_hbm.at[srcv.at[j + 2]], rows_a, sem_a)
            pltpu.make_async_copy(m_hbm.at[srcv.at[j + 1]], rows_b,
                                  sem_b).wait()
            pltpu.sync_copy(rows_b, agg_sh.at[dstv.at[j + 1]], add=True)
            return carry

        lax.fori_loop(0, TPB // 2, eb2, 0)
        pltpu.make_async_copy(m_hbm.at[srcv.at[TPB]], rows_a, sem_a).wait()
        plsc.subcore_barrier()

        pltpu.sync_copy(agg_sh.at[pl.ds(sid * RPT, RPT)],
                        out_hbm.at[cid, pl.ds(sid * RPT, RPT)])

    return pl.kernel(
        body,
        out_type=jax.ShapeDtypeStruct((NC, NP, D), _f32),
        mesh=_mesh,
        scratch_types=[
            pltpu.VMEM((TPB + 1, EB), jnp.int32),
            pltpu.VMEM((TPB, EB), jnp.int32),
            pltpu.VMEM((EB, D), _f32),
            pltpu.VMEM((EB, D), _f32),
            pltpu.VMEM_SHARED((NP, D), _f32),
            pltpu.SemaphoreType.DMA,
            pltpu.SemaphoreType.DMA,
        ],
        compiler_params=pltpu.CompilerParams(needs_layout_passes=False, use_tc_tiling_on_sc=False),
    )


_agg = _make_agg()


BN = 1024


def _scale_tc(x_ref, ns_ref, o_ref):
    o_ref[...] = x_ref[...] * ns_ref[...]


def _mm1_tc(agg_ref, ns_ref, nd_ref, w_ref, b_ref, o_ref):
    a = agg_ref[0] + agg_ref[1]
    p = jnp.dot(a, w_ref[...], preferred_element_type=_f32)
    o_ref[...] = jnp.maximum(nd_ref[...] * p + b_ref[...], 0.0) * ns_ref[...]


def _mm2_tc(agg_ref, nd_ref, w_ref, b_ref, wc_ref, bc_ref, h_ref, l_ref):
    a = agg_ref[0] + agg_ref[1]
    h = nd_ref[...] * jnp.dot(a, w_ref[...], preferred_element_type=_f32) \
        + b_ref[...]
    h_ref[...] = h
    l_ref[...] = jnp.dot(jnp.maximum(h, 0.0), wc_ref[...],
                         preferred_element_type=_f32) + bc_ref[...]


def _scale_call(xp, ns_col):
    return pl.pallas_call(
        _scale_tc,
        grid=(NP // BN,),
        in_specs=[
            pl.BlockSpec((BN, D), lambda i: (i, 0)),
            pl.BlockSpec((BN, 1), lambda i: (i, 0)),
        ],
        out_specs=pl.BlockSpec((BN, D), lambda i: (i, 0)),
        out_shape=jax.ShapeDtypeStruct((NP, D), _f32),
    )(xp, ns_col)


def _mm1_call(aggp, ns_col, nd_col, W, b_row):
    return pl.pallas_call(
        _mm1_tc,
        grid=(NP // BN,),
        in_specs=[
            pl.BlockSpec((NC, BN, D), lambda i: (0, i, 0)),
            pl.BlockSpec((BN, 1), lambda i: (i, 0)),
            pl.BlockSpec((BN, 1), lambda i: (i, 0)),
            pl.BlockSpec((D, D), lambda i: (0, 0)),
            pl.BlockSpec((1, D), lambda i: (0, 0)),
        ],
        out_specs=pl.BlockSpec((BN, D), lambda i: (i, 0)),
        out_shape=jax.ShapeDtypeStruct((NP, D), _f32),
    )(aggp, ns_col, nd_col, W, b_row)


def _mm2_call(aggp, nd_col, W, b_row, Wc, bc_row):
    return pl.pallas_call(
        _mm2_tc,
        grid=(NP // BN,),
        in_specs=[
            pl.BlockSpec((NC, BN, D), lambda i: (0, i, 0)),
            pl.BlockSpec((BN, 1), lambda i: (i, 0)),
            pl.BlockSpec((D, D), lambda i: (0, 0)),
            pl.BlockSpec((1, D), lambda i: (0, 0)),
            pl.BlockSpec((D, C), lambda i: (0, 0)),
            pl.BlockSpec((1, C), lambda i: (0, 0)),
        ],
        out_specs=[
            pl.BlockSpec((BN, D), lambda i: (i, 0)),
            pl.BlockSpec((BN, C), lambda i: (i, 0)),
        ],
        out_shape=[
            jax.ShapeDtypeStruct((NP, D), _f32),
            jax.ShapeDtypeStruct((NP, C), _f32),
        ],
    )(aggp, nd_col, W, b_row, Wc, bc_row)


@jax.jit
def kernel(g, x, W1, b1, W2, b2, Wc, bc):
    pad_nodes = N + jax.lax.rem(jnp.arange(EP - E, dtype=jnp.int32),
                                jnp.int32(NP - N))
    pad = jnp.stack([pad_nodes, pad_nodes])
    gidx = jnp.concatenate([g, pad], axis=1).reshape(2, NW * TPB, EB)
    xp = jnp.pad(x, ((0, NP - N), (0, 0)))

    norms = _degnorm(gidx)
    ns_col = norms[0].reshape(NP, 1)
    nd_col = norms[1].reshape(NP, 1)

    m1 = _scale_call(xp, ns_col)
    aggp1 = _agg(gidx, m1)
    m2 = _mm1_call(aggp1, ns_col, nd_col, W1, b1.reshape(1, D))
    aggp2 = _agg(gidx, m2)
    h, logits = _mm2_call(aggp2, nd_col, W2, b2.reshape(1, D),
                          Wc, bc.reshape(1, C))
    return h[:N], logits[:N]


# speedup vs baseline: 11.2538x; 1.0852x over previous
import functools
import jax
import jax.numpy as jnp
from jax import lax
from jax.experimental import pallas as pl
from jax.experimental.pallas import tpu as pltpu
from jax.experimental.pallas import tpu_sc as plsc

N = 10000
D = 128
C = 64
E = 320000

NC = 2
NS = 16
NW = NC * NS

NP = 10240
RPT = NP // NS
EB = 32
TPB = 320
KB = 4
EPT = TPB * EB
EP = NW * EPT
ERT = EP // NS // EB
DUMP = NP - 1

_mesh = plsc.VectorSubcoreMesh(core_axis_name="c", subcore_axis_name="s")
_f32 = jnp.float32


def _quake_rsqrt16(s):
    ii = plsc.bitcast(s, jnp.int32)
    ii = 0x5F3759DF - lax.shift_right_logical(ii, 1)
    y = plsc.bitcast(ii, _f32)
    hs = 0.5 * s
    for _ in range(3):
        y = y * (1.5 - hs * y * y)
    return jnp.where(s > 0.0, y, jnp.zeros((16,), _f32))


def _degnorm_body(gidx, norms_out, idx_v, deg_v, stage, acc_v, sh):
    cid = lax.axis_index("c")
    sid = lax.axis_index("s")

    pltpu.sync_copy(gidx.at[cid, pl.ds(sid * ERT, ERT)], idx_v)

    zeros16 = jnp.zeros((16,), _f32)

    def zb(i, carry):
        deg_v[pl.ds(i * 16, 16)] = zeros16
        return carry

    lax.fori_loop(0, NP // 16, zb, 0)

    ones16 = jnp.ones((16,), _f32)

    def sb(r, carry):
        for c in range(EB // 16):
            plsc.addupdate_scatter(deg_v, [idx_v[r, pl.ds(c * 16, 16)]],
                                   ones16)
        return carry

    lax.fori_loop(0, ERT, sb, 0)

    pltpu.sync_copy(deg_v, sh.at[sid])
    plsc.subcore_barrier()

    nb = sid * RPT
    for k in range(NS):
        pltpu.sync_copy(sh.at[k, pl.ds(nb, RPT)], stage.at[k])

    def rb(i, carry):
        s = stage[0, pl.ds(i * 16, 16)]
        for k in range(1, NS):
            s = s + stage[k, pl.ds(i * 16, 16)]
        acc_v[pl.ds(i * 16, 16)] = _quake_rsqrt16(s)
        return carry

    lax.fori_loop(0, RPT // 16, rb, 0)

    pltpu.sync_copy(acc_v, norms_out.at[cid, pl.ds(nb, RPT)])


@functools.partial(
    pl.kernel,
    out_type=jax.ShapeDtypeStruct((NC, NP), _f32),
    mesh=_mesh,
    scratch_types=[
        pltpu.VMEM((ERT, EB), jnp.int32),
        pltpu.VMEM((NP,), _f32),
        pltpu.VMEM((NS, RPT), _f32),
        pltpu.VMEM((RPT,), _f32),
        pltpu.VMEM_SHARED((NS, NP), _f32),
    ],
    compiler_params=pltpu.CompilerParams(needs_layout_passes=False, use_tc_tiling_on_sc=False),
)
def _degnorm(*args):
    _degnorm_body(*args)


def _make_agg():
    def body(gidx, m_hbm, out_hbm, srcv, dstv,
             rows0, rows1, rows2, rows3,
             agg_sh, gs0, gs1, gs2, gs3, ss0, ss1, ss2, ss3):
        cid = lax.axis_index("c")
        sid = lax.axis_index("s")
        wid = cid * NS + sid
        rows = (rows0, rows1, rows2, rows3)
        gsem = (gs0, gs1, gs2, gs3)
        ssem = (ss0, ss1, ss2, ss3)

        pltpu.sync_copy(gidx.at[0, pl.ds(wid * TPB, TPB)],
                        srcv.at[pl.ds(0, TPB)])
        pltpu.sync_copy(gidx.at[0, pl.ds(wid * TPB, 1)],
                        srcv.at[pl.ds(TPB, 1)])
        pltpu.sync_copy(gidx.at[0, pl.ds(wid * TPB, 1)],
                        srcv.at[pl.ds(TPB + 1, 1)])
        pltpu.sync_copy(gidx.at[1, pl.ds(wid * TPB, TPB)], dstv)

        zeros16 = jnp.zeros((16,), _f32)

        def zb(i, carry):
            for c in range(D // 16):
                for rb in rows:
                    rb[i, pl.ds(c * 16, 16)] = zeros16
            return carry

        lax.fori_loop(0, EB, zb, 0)

        for t in range(RPT // EB):
            pltpu.sync_copy(rows0, agg_sh.at[pl.ds(sid * RPT + t * EB, EB)])
        plsc.subcore_barrier()

        pltpu.async_copy(m_hbm.at[srcv.at[0]], rows0, gsem[0])
        pltpu.async_copy(m_hbm.at[srcv.at[1]], rows1, gsem[1])
        pltpu.async_copy(rows2, agg_sh.at[dstv.at[0]], ssem[2], add=True)
        pltpu.async_copy(rows3, agg_sh.at[dstv.at[0]], ssem[3], add=True)

        def slot(j, k, kn):
            pltpu.make_async_copy(rows[kn], agg_sh.at[dstv.at[0]],
                                  ssem[kn]).wait()
            pltpu.async_copy(m_hbm.at[srcv.at[j + 2]], rows[kn], gsem[kn])
            pltpu.make_async_copy(m_hbm.at[srcv.at[j]], rows[k],
                                  gsem[k]).wait()
            pltpu.async_copy(rows[k], agg_sh.at[dstv.at[j]], ssem[k],
                             add=True)

        def eb4(t, carry):
            j = 4 * t
            slot(j, 0, 2)
            slot(j + 1, 1, 3)
            slot(j + 2, 2, 0)
            slot(j + 3, 3, 1)
            return carry

        lax.fori_loop(0, TPB // KB, eb4, 0)
        pltpu.make_async_copy(rows[2], agg_sh.at[dstv.at[0]], ssem[2]).wait()
        pltpu.make_async_copy(rows[3], agg_sh.at[dstv.at[0]], ssem[3]).wait()
        pltpu.make_async_copy(m_hbm.at[srcv.at[0]], rows[0], gsem[0]).wait()
        pltpu.make_async_copy(m_hbm.at[srcv.at[1]], rows[1], gsem[1]).wait()
        plsc.subcore_barrier()

        pltpu.sync_copy(agg_sh.at[pl.ds(sid * RPT, RPT)],
                        out_hbm.at[cid, pl.ds(sid * RPT, RPT)])

    return pl.kernel(
        body,
        out_type=jax.ShapeDtypeStruct((NC, NP, D), _f32),
        mesh=_mesh,
        scratch_types=[
            pltpu.VMEM((TPB + 2, EB), jnp.int32),
            pltpu.VMEM((TPB, EB), jnp.int32),
            pltpu.VMEM((EB, D), _f32),
            pltpu.VMEM((EB, D), _f32),
            pltpu.VMEM((EB, D), _f32),
            pltpu.VMEM((EB, D), _f32),
            pltpu.VMEM_SHARED((NP, D), _f32),
            pltpu.SemaphoreType.DMA,
            pltpu.SemaphoreType.DMA,
            pltpu.SemaphoreType.DMA,
            pltpu.SemaphoreType.DMA,
            pltpu.SemaphoreType.DMA,
            pltpu.SemaphoreType.DMA,
            pltpu.SemaphoreType.DMA,
            pltpu.SemaphoreType.DMA,
        ],
        compiler_params=pltpu.CompilerParams(needs_layout_passes=False, use_tc_tiling_on_sc=False),
    )


_agg = _make_agg()


BN = 1024


def _scale_tc(x_ref, ns_ref, o_ref):
    o_ref[...] = x_ref[...] * ns_ref[...]


def _mm1_tc(agg_ref, ns_ref, nd_ref, w_ref, b_ref, o_ref):
    a = agg_ref[0] + agg_ref[1]
    p = jnp.dot(a, w_ref[...], preferred_element_type=_f32)
    o_ref[...] = jnp.maximum(nd_ref[...] * p + b_ref[...], 0.0) * ns_ref[...]


def _mm2_tc(agg_ref, nd_ref, w_ref, b_ref, wc_ref, bc_ref, h_ref, l_ref):
    a = agg_ref[0] + agg_ref[1]
    h = nd_ref[...] * jnp.dot(a, w_ref[...], preferred_element_type=_f32) \
        + b_ref[...]
    h_ref[...] = h
    l_ref[...] = jnp.dot(jnp.maximum(h, 0.0), wc_ref[...],
                         preferred_element_type=_f32) + bc_ref[...]


def _scale_call(xp, ns_col):
    return pl.pallas_call(
        _scale_tc,
        grid=(NP // BN,),
        in_specs=[
            pl.BlockSpec((BN, D), lambda i: (i, 0)),
            pl.BlockSpec((BN, 1), lambda i: (i, 0)),
        ],
        out_specs=pl.BlockSpec((BN, D), lambda i: (i, 0)),
        out_shape=jax.ShapeDtypeStruct((NP, D), _f32),
    )(xp, ns_col)


def _mm1_call(aggp, ns_col, nd_col, W, b_row):
    return pl.pallas_call(
        _mm1_tc,
        grid=(NP // BN,),
        in_specs=[
            pl.BlockSpec((NC, BN, D), lambda i: (0, i, 0)),
            pl.BlockSpec((BN, 1), lambda i: (i, 0)),
            pl.BlockSpec((BN, 1), lambda i: (i, 0)),
            pl.BlockSpec((D, D), lambda i: (0, 0)),
            pl.BlockSpec((1, D), lambda i: (0, 0)),
        ],
        out_specs=pl.BlockSpec((BN, D), lambda i: (i, 0)),
        out_shape=jax.ShapeDtypeStruct((NP, D), _f32),
    )(aggp, ns_col, nd_col, W, b_row)


def _mm2_call(aggp, nd_col, W, b_row, Wc, bc_row):
    return pl.pallas_call(
        _mm2_tc,
        grid=(NP // BN,),
        in_specs=[
            pl.BlockSpec((NC, BN, D), lambda i: (0, i, 0)),
            pl.BlockSpec((BN, 1), lambda i: (i, 0)),
            pl.BlockSpec((D, D), lambda i: (0, 0)),
            pl.BlockSpec((1, D), lambda i: (0, 0)),
            pl.BlockSpec((D, C), lambda i: (0, 0)),
            pl.BlockSpec((1, C), lambda i: (0, 0)),
        ],
        out_specs=[
            pl.BlockSpec((BN, D), lambda i: (i, 0)),
            pl.BlockSpec((BN, C), lambda i: (i, 0)),
        ],
        out_shape=[
            jax.ShapeDtypeStruct((NP, D), _f32),
            jax.ShapeDtypeStruct((NP, C), _f32),
        ],
    )(aggp, nd_col, W, b_row, Wc, bc_row)


@jax.jit
def kernel(g, x, W1, b1, W2, b2, Wc, bc):
    pad_nodes = N + jax.lax.rem(jnp.arange(EP - E, dtype=jnp.int32),
                                jnp.int32(NP - N))
    pad = jnp.stack([pad_nodes, pad_nodes])
    gidx = jnp.concatenate([g, pad], axis=1).reshape(2, NW * TPB, EB)
    xp = jnp.pad(x, ((0, NP - N), (0, 0)))

    norms = _degnorm(gidx)
    ns_col = norms[0].reshape(NP, 1)
    nd_col = norms[1].reshape(NP, 1)

    m1 = _scale_call(xp, ns_col)
    aggp1 = _agg(gidx, m1)
    m2 = _mm1_call(aggp1, ns_col, nd_col, W1, b1.reshape(1, D))
    aggp2 = _agg(gidx, m2)
    h, logits = _mm2_call(aggp2, nd_col, W2, b2.reshape(1, D),
                          Wc, bc.reshape(1, C))
    return h[:N], logits[:N]


# speedup vs baseline: 11.4666x; 1.0189x over previous
import functools
import jax
import jax.numpy as jnp
from jax import lax
from jax.experimental import pallas as pl
from jax.experimental.pallas import tpu as pltpu
from jax.experimental.pallas import tpu_sc as plsc

N = 10000
D = 128
C = 64
E = 320000

NC = 2
NS = 16
NW = NC * NS

NP = 10240
RPT = NP // NS
EB = 32
TPB = 320
KB = 4
EPT = TPB * EB
EP = NW * EPT
ERT = EP // NS // EB
DUMP = NP - 1

_mesh = plsc.VectorSubcoreMesh(core_axis_name="c", subcore_axis_name="s")
_f32 = jnp.float32


def _quake_rsqrt16(s):
    ii = plsc.bitcast(s, jnp.int32)
    ii = 0x5F3759DF - lax.shift_right_logical(ii, 1)
    y = plsc.bitcast(ii, _f32)
    hs = 0.5 * s
    for _ in range(3):
        y = y * (1.5 - hs * y * y)
    return jnp.where(s > 0.0, y, jnp.zeros((16,), _f32))


def _degnorm_body(gidx, norms_out, idx_v, deg_v, stage, acc_v, sh):
    cid = lax.axis_index("c")
    sid = lax.axis_index("s")

    pltpu.sync_copy(gidx.at[cid, pl.ds(sid * ERT, ERT)], idx_v)

    zeros16 = jnp.zeros((16,), _f32)

    def zb(i, carry):
        deg_v[pl.ds(i * 16, 16)] = zeros16
        return carry

    lax.fori_loop(0, NP // 16, zb, 0)

    ones16 = jnp.ones((16,), _f32)

    def sb(r, carry):
        for c in range(EB // 16):
            plsc.addupdate_scatter(deg_v, [idx_v[r, pl.ds(c * 16, 16)]],
                                   ones16)
        return carry

    lax.fori_loop(0, ERT, sb, 0)

    pltpu.sync_copy(deg_v, sh.at[sid])
    plsc.subcore_barrier()

    nb = sid * RPT
    for k in range(NS):
        pltpu.sync_copy(sh.at[k, pl.ds(nb, RPT)], stage.at[k])

    def rb(i, carry):
        s = stage[0, pl.ds(i * 16, 16)]
        for k in range(1, NS):
            s = s + stage[k, pl.ds(i * 16, 16)]
        acc_v[pl.ds(i * 16, 16)] = _quake_rsqrt16(s)
        return carry

    lax.fori_loop(0, RPT // 16, rb, 0)

    pltpu.sync_copy(acc_v, norms_out.at[cid, pl.ds(nb, RPT)])


@functools.partial(
    pl.kernel,
    out_type=jax.ShapeDtypeStruct((NC, NP), _f32),
    mesh=_mesh,
    scratch_types=[
        pltpu.VMEM((ERT, EB), jnp.int32),
        pltpu.VMEM((NP,), _f32),
        pltpu.VMEM((NS, RPT), _f32),
        pltpu.VMEM((RPT,), _f32),
        pltpu.VMEM_SHARED((NS, NP), _f32),
    ],
    compiler_params=pltpu.CompilerParams(needs_layout_passes=False, use_tc_tiling_on_sc=False),
)
def _degnorm(*args):
    _degnorm_body(*args)


def _make_agg():
    def body(gidx, m_hbm, out_hbm, srcv, dstv,
             rows0, rows1, rows2, rows3,
             agg_sh, gs0, gs1, gs2, gs3, ss0, ss1, ss2, ss3):
        cid = lax.axis_index("c")
        sid = lax.axis_index("s")
        wid = cid * NS + sid
        rows = (rows0, rows1, rows2, rows3)
        gsem = (gs0, gs1, gs2, gs3)
        ssem = (ss0, ss1, ss2, ss3)

        pltpu.sync_copy(gidx.at[0, pl.ds(wid * TPB, TPB)],
                        srcv.at[pl.ds(0, TPB)])
        pltpu.sync_copy(gidx.at[0, pl.ds(wid * TPB, 1)],
                        srcv.at[pl.ds(TPB, 1)])
        pltpu.sync_copy(gidx.at[0, pl.ds(wid * TPB, 1)],
                        srcv.at[pl.ds(TPB + 1, 1)])
        pltpu.sync_copy(gidx.at[1, pl.ds(wid * TPB, TPB)], dstv)

        zeros16 = jnp.zeros((16,), _f32)

        def zb(i, carry):
            for c in range(D // 16):
                for rb in rows:
                    rb[i, pl.ds(c * 16, 16)] = zeros16
            return carry

        lax.fori_loop(0, EB, zb, 0)

        for t in range(RPT // EB):
            pltpu.sync_copy(rows0, agg_sh.at[pl.ds(sid * RPT + t * EB, EB)])
        plsc.subcore_barrier()

        pltpu.async_copy(m_hbm.at[srcv.at[0]], rows0, gsem[0])
        pltpu.async_copy(m_hbm.at[srcv.at[1]], rows1, gsem[1])
        pltpu.async_copy(rows2, agg_sh.at[dstv.at[0]], ssem[2], add=True)
        pltpu.async_copy(rows3, agg_sh.at[dstv.at[0]], ssem[3], add=True)

        def slot(j, k, kn):
            pltpu.make_async_copy(rows[kn], agg_sh.at[dstv.at[0]],
                                  ssem[kn]).wait()
            pltpu.async_copy(m_hbm.at[srcv.at[j + 2]], rows[kn], gsem[kn])
            pltpu.make_async_copy(m_hbm.at[srcv.at[j]], rows[k],
                                  gsem[k]).wait()
            pltpu.async_copy(rows[k], agg_sh.at[dstv.at[j]], ssem[k],
                             add=True)

        def eb4(t, carry):
            j = 4 * t
            slot(j, 0, 2)
            slot(j + 1, 1, 3)
            slot(j + 2, 2, 0)
            slot(j + 3, 3, 1)
            return carry

        lax.fori_loop(0, TPB // KB, eb4, 0)
        pltpu.make_async_copy(rows[2], agg_sh.at[dstv.at[0]], ssem[2]).wait()
        pltpu.make_async_copy(rows[3], agg_sh.at[dstv.at[0]], ssem[3]).wait()
        pltpu.make_async_copy(m_hbm.at[srcv.at[0]], rows[0], gsem[0]).wait()
        pltpu.make_async_copy(m_hbm.at[srcv.at[1]], rows[1], gsem[1]).wait()
        plsc.subcore_barrier()

        pltpu.sync_copy(agg_sh.at[pl.ds(sid * RPT, RPT)],
                        out_hbm.at[cid, pl.ds(sid * RPT, RPT)])

    return pl.kernel(
        body,
        out_type=jax.ShapeDtypeStruct((NC, NP, D), _f32),
        mesh=_mesh,
        scratch_types=[
            pltpu.VMEM((TPB + 2, EB), jnp.int32),
            pltpu.VMEM((TPB, EB), jnp.int32),
            pltpu.VMEM((EB, D), _f32),
            pltpu.VMEM((EB, D), _f32),
            pltpu.VMEM((EB, D), _f32),
            pltpu.VMEM((EB, D), _f32),
            pltpu.VMEM_SHARED((NP, D), _f32),
            pltpu.SemaphoreType.DMA,
            pltpu.SemaphoreType.DMA,
            pltpu.SemaphoreType.DMA,
            pltpu.SemaphoreType.DMA,
            pltpu.SemaphoreType.DMA,
            pltpu.SemaphoreType.DMA,
            pltpu.SemaphoreType.DMA,
            pltpu.SemaphoreType.DMA,
        ],
        compiler_params=pltpu.CompilerParams(needs_layout_passes=False, use_tc_tiling_on_sc=False),
    )


_agg = _make_agg()


BN = 1024


def _scale_tc(x_ref, ns_ref, o_ref):
    o_ref[...] = x_ref[...] * ns_ref[...]


def _mm1_tc(agg_ref, ns_ref, nd_ref, w_ref, b_ref, o_ref):
    a = agg_ref[0] + agg_ref[1]
    p = jnp.dot(a, w_ref[...], preferred_element_type=_f32)
    o_ref[...] = jnp.maximum(nd_ref[...] * p + b_ref[...], 0.0) * ns_ref[...]


def _mm2_tc(agg_ref, nd_ref, w_ref, b_ref, wc_ref, bc_ref, h_ref, l_ref):
    a = agg_ref[0] + agg_ref[1]
    h = nd_ref[...] * jnp.dot(a, w_ref[...], preferred_element_type=_f32) \
        + b_ref[...]
    h_ref[...] = h
    l_ref[...] = jnp.dot(jnp.maximum(h, 0.0), wc_ref[...],
                         preferred_element_type=_f32) + bc_ref[...]


def _scale_call(x, ns_col):
    return pl.pallas_call(
        _scale_tc,
        grid=(NP // BN,),
        in_specs=[
            pl.BlockSpec((BN, D), lambda i: (i, 0)),
            pl.BlockSpec((BN, 1), lambda i: (i, 0)),
        ],
        out_specs=pl.BlockSpec((BN, D), lambda i: (i, 0)),
        out_shape=jax.ShapeDtypeStruct((NP, D), _f32),
    )(x, ns_col)


def _mm1_call(aggp, ns_col, nd_col, W, b_row):
    return pl.pallas_call(
        _mm1_tc,
        grid=(NP // BN,),
        in_specs=[
            pl.BlockSpec((NC, BN, D), lambda i: (0, i, 0)),
            pl.BlockSpec((BN, 1), lambda i: (i, 0)),
            pl.BlockSpec((BN, 1), lambda i: (i, 0)),
            pl.BlockSpec((D, D), lambda i: (0, 0)),
            pl.BlockSpec((1, D), lambda i: (0, 0)),
        ],
        out_specs=pl.BlockSpec((BN, D), lambda i: (i, 0)),
        out_shape=jax.ShapeDtypeStruct((NP, D), _f32),
    )(aggp, ns_col, nd_col, W, b_row)


def _mm2_call(aggp, nd_col, W, b_row, Wc, bc_row):
    return pl.pallas_call(
        _mm2_tc,
        grid=(NP // BN,),
        in_specs=[
            pl.BlockSpec((NC, BN, D), lambda i: (0, i, 0)),
            pl.BlockSpec((BN, 1), lambda i: (i, 0)),
            pl.BlockSpec((D, D), lambda i: (0, 0)),
            pl.BlockSpec((1, D), lambda i: (0, 0)),
            pl.BlockSpec((D, C), lambda i: (0, 0)),
            pl.BlockSpec((1, C), lambda i: (0, 0)),
        ],
        out_specs=[
            pl.BlockSpec((BN, D), lambda i: (i, 0)),
            pl.BlockSpec((BN, C), lambda i: (i, 0)),
        ],
        out_shape=[
            jax.ShapeDtypeStruct((N, D), _f32),
            jax.ShapeDtypeStruct((N, C), _f32),
        ],
    )(aggp, nd_col, W, b_row, Wc, bc_row)


@jax.jit
def kernel(g, x, W1, b1, W2, b2, Wc, bc):
    pad_nodes = N + jax.lax.rem(jnp.arange(EP - E, dtype=jnp.int32),
                                jnp.int32(NP - N))
    pad = jnp.stack([pad_nodes, pad_nodes])
    gidx = jnp.concatenate([g, pad], axis=1).reshape(2, NW * TPB, EB)

    norms = _degnorm(gidx)
    ns_col = norms[0].reshape(NP, 1)
    nd_col = norms[1].reshape(NP, 1)

    m1 = _scale_call(x, ns_col)
    aggp1 = _agg(gidx, m1)
    m2 = _mm1_call(aggp1, ns_col, nd_col, W1, b1.reshape(1, D))
    aggp2 = _agg(gidx, m2)
    h, logits = _mm2_call(aggp2, nd_col, W2, b2.reshape(1, D),
                          Wc, bc.reshape(1, C))
    return h, logits


# speedup vs baseline: 12.1244x; 1.0574x over previous
import functools
import jax
import jax.numpy as jnp
from jax import lax
from jax.experimental import pallas as pl
from jax.experimental.pallas import tpu as pltpu
from jax.experimental.pallas import tpu_sc as plsc

N = 10000
D = 128
C = 64
E = 320000

NC = 2
NS = 16
NW = NC * NS

NP = 10240
RPT = NP // NS
EB = 40
TPB = 256
KB = 4
EPT = TPB * EB
EP = NW * EPT
ERT = EP // NS // EB
DUMP = NP - 1

_mesh = plsc.VectorSubcoreMesh(core_axis_name="c", subcore_axis_name="s")
_f32 = jnp.float32


def _quake_rsqrt16(s):
    ii = plsc.bitcast(s, jnp.int32)
    ii = 0x5F3759DF - lax.shift_right_logical(ii, 1)
    y = plsc.bitcast(ii, _f32)
    hs = 0.5 * s
    for _ in range(3):
        y = y * (1.5 - hs * y * y)
    return jnp.where(s > 0.0, y, jnp.zeros((16,), _f32))


def _degnorm_body(gidx, norms_out, idx_v, deg_v, stage, acc_v, sh):
    cid = lax.axis_index("c")
    sid = lax.axis_index("s")

    pltpu.sync_copy(gidx.at[cid, pl.ds(sid * ERT, ERT)], idx_v)

    zeros16 = jnp.zeros((16,), _f32)

    def zb(i, carry):
        deg_v[pl.ds(i * 16, 16)] = zeros16
        return carry

    lax.fori_loop(0, NP // 16, zb, 0)

    ones16 = jnp.ones((16,), _f32)

    def sb(r, carry):
        for c in range(EB // 16):
            plsc.addupdate_scatter(deg_v, [idx_v[r, pl.ds(c * 16, 16)]],
                                   ones16)
        return carry

    lax.fori_loop(0, ERT, sb, 0)

    pltpu.sync_copy(deg_v, sh.at[sid])
    plsc.subcore_barrier()

    nb = sid * RPT
    for k in range(NS):
        pltpu.sync_copy(sh.at[k, pl.ds(nb, RPT)], stage.at[k])

    def rb(i, carry):
        s = stage[0, pl.ds(i * 16, 16)]
        for k in range(1, NS):
            s = s + stage[k, pl.ds(i * 16, 16)]
        acc_v[pl.ds(i * 16, 16)] = _quake_rsqrt16(s)
        return carry

    lax.fori_loop(0, RPT // 16, rb, 0)

    pltpu.sync_copy(acc_v, norms_out.at[cid, pl.ds(nb, RPT)])


@functools.partial(
    pl.kernel,
    out_type=jax.ShapeDtypeStruct((NC, NP), _f32),
    mesh=_mesh,
    scratch_types=[
        pltpu.VMEM((ERT, EB), jnp.int32),
        pltpu.VMEM((NP,), _f32),
        pltpu.VMEM((NS, RPT), _f32),
        pltpu.VMEM((RPT,), _f32),
        pltpu.VMEM_SHARED((NS, NP), _f32),
    ],
    compiler_params=pltpu.CompilerParams(needs_layout_passes=False, use_tc_tiling_on_sc=False),
)
def _degnorm(*args):
    _degnorm_body(*args)


def _make_agg():
    def body(gidx, m_hbm, out_hbm, srcv, dstv,
             rows0, rows1, rows2, rows3,
             agg_sh, gs0, gs1, gs2, gs3, ss0, ss1, ss2, ss3):
        cid = lax.axis_index("c")
        sid = lax.axis_index("s")
        wid = cid * NS + sid
        rows = (rows0, rows1, rows2, rows3)
        gsem = (gs0, gs1, gs2, gs3)
        ssem = (ss0, ss1, ss2, ss3)

        pltpu.sync_copy(gidx.at[0, pl.ds(wid * TPB, TPB)],
                        srcv.at[pl.ds(0, TPB)])
        pltpu.sync_copy(gidx.at[0, pl.ds(wid * TPB, 1)],
                        srcv.at[pl.ds(TPB, 1)])
        pltpu.sync_copy(gidx.at[0, pl.ds(wid * TPB, 1)],
                        srcv.at[pl.ds(TPB + 1, 1)])
        pltpu.sync_copy(gidx.at[1, pl.ds(wid * TPB, TPB)], dstv)

        zeros16 = jnp.zeros((16,), _f32)

        def zb(i, carry):
            for c in range(D // 16):
                for rb in rows:
                    rb[i, pl.ds(c * 16, 16)] = zeros16
            return carry

        lax.fori_loop(0, EB, zb, 0)

        for t in range(RPT // EB):
            pltpu.sync_copy(rows0, agg_sh.at[pl.ds(sid * RPT + t * EB, EB)])
        plsc.subcore_barrier()

        pltpu.async_copy(m_hbm.at[srcv.at[0]], rows0, gsem[0])
        pltpu.async_copy(m_hbm.at[srcv.at[1]], rows1, gsem[1])
        pltpu.async_copy(rows2, agg_sh.at[dstv.at[0]], ssem[2], add=True)
        pltpu.async_copy(rows3, agg_sh.at[dstv.at[0]], ssem[3], add=True)

        def slot(j, k, kn):
            pltpu.make_async_copy(rows[kn], agg_sh.at[dstv.at[0]],
                                  ssem[kn]).wait()
            pltpu.async_copy(m_hbm.at[srcv.at[j + 2]], rows[kn], gsem[kn])
            pltpu.make_async_copy(m_hbm.at[srcv.at[j]], rows[k],
                                  gsem[k]).wait()
            pltpu.async_copy(rows[k], agg_sh.at[dstv.at[j]], ssem[k],
                             add=True)

        def eb4(t, carry):
            j = 4 * t
            slot(j, 0, 2)
            slot(j + 1, 1, 3)
            slot(j + 2, 2, 0)
            slot(j + 3, 3, 1)
            return carry

        lax.fori_loop(0, TPB // KB, eb4, 0)
        pltpu.make_async_copy(rows[2], agg_sh.at[dstv.at[0]], ssem[2]).wait()
        pltpu.make_async_copy(rows[3], agg_sh.at[dstv.at[0]], ssem[3]).wait()
        pltpu.make_async_copy(m_hbm.at[srcv.at[0]], rows[0], gsem[0]).wait()
        pltpu.make_async_copy(m_hbm.at[srcv.at[1]], rows[1], gsem[1]).wait()
        plsc.subcore_barrier()

        pltpu.sync_copy(agg_sh.at[pl.ds(sid * RPT, RPT)],
                        out_hbm.at[cid, pl.ds(sid * RPT, RPT)])

    return pl.kernel(
        body,
        out_type=jax.ShapeDtypeStruct((NC, NP, D), _f32),
        mesh=_mesh,
        scratch_types=[
            pltpu.VMEM((TPB + 2, EB), jnp.int32),
            pltpu.VMEM((TPB, EB), jnp.int32),
            pltpu.VMEM((EB, D), _f32),
            pltpu.VMEM((EB, D), _f32),
            pltpu.VMEM((EB, D), _f32),
            pltpu.VMEM((EB, D), _f32),
            pltpu.VMEM_SHARED((NP, D), _f32),
            pltpu.SemaphoreType.DMA,
            pltpu.SemaphoreType.DMA,
            pltpu.SemaphoreType.DMA,
            pltpu.SemaphoreType.DMA,
            pltpu.SemaphoreType.DMA,
            pltpu.SemaphoreType.DMA,
            pltpu.SemaphoreType.DMA,
            pltpu.SemaphoreType.DMA,
        ],
        compiler_params=pltpu.CompilerParams(needs_layout_passes=False, use_tc_tiling_on_sc=False),
    )


_agg = _make_agg()


BN = 1024


def _scale_tc(x_ref, ns_ref, o_ref):
    o_ref[...] = x_ref[...] * ns_ref[...]


def _mm1_tc(agg_ref, ns_ref, nd_ref, w_ref, b_ref, o_ref):
    a = agg_ref[0] + agg_ref[1]
    p = jnp.dot(a, w_ref[...], preferred_element_type=_f32)
    o_ref[...] = jnp.maximum(nd_ref[...] * p + b_ref[...], 0.0) * ns_ref[...]


def _mm2_tc(agg_ref, nd_ref, w_ref, b_ref, wc_ref, bc_ref, h_ref, l_ref):
    a = agg_ref[0] + agg_ref[1]
    h = nd_ref[...] * jnp.dot(a, w_ref[...], preferred_element_type=_f32) \
        + b_ref[...]
    h_ref[...] = h
    l_ref[...] = jnp.dot(jnp.maximum(h, 0.0), wc_ref[...],
                         preferred_element_type=_f32) + bc_ref[...]


def _scale_call(x, ns_col):
    return pl.pallas_call(
        _scale_tc,
        grid=(NP // BN,),
        in_specs=[
            pl.BlockSpec((BN, D), lambda i: (i, 0)),
            pl.BlockSpec((BN, 1), lambda i: (i, 0)),
        ],
        out_specs=pl.BlockSpec((BN, D), lambda i: (i, 0)),
        out_shape=jax.ShapeDtypeStruct((NP, D), _f32),
    )(x, ns_col)


def _mm1_call(aggp, ns_col, nd_col, W, b_row):
    return pl.pallas_call(
        _mm1_tc,
        grid=(NP // BN,),
        in_specs=[
            pl.BlockSpec((NC, BN, D), lambda i: (0, i, 0)),
            pl.BlockSpec((BN, 1), lambda i: (i, 0)),
            pl.BlockSpec((BN, 1), lambda i: (i, 0)),
            pl.BlockSpec((D, D), lambda i: (0, 0)),
            pl.BlockSpec((1, D), lambda i: (0, 0)),
        ],
        out_specs=pl.BlockSpec((BN, D), lambda i: (i, 0)),
        out_shape=jax.ShapeDtypeStruct((NP, D), _f32),
    )(aggp, ns_col, nd_col, W, b_row)


def _mm2_call(aggp, nd_col, W, b_row, Wc, bc_row):
    return pl.pallas_call(
        _mm2_tc,
        grid=(NP // BN,),
        in_specs=[
            pl.BlockSpec((NC, BN, D), lambda i: (0, i, 0)),
            pl.BlockSpec((BN, 1), lambda i: (i, 0)),
            pl.BlockSpec((D, D), lambda i: (0, 0)),
            pl.BlockSpec((1, D), lambda i: (0, 0)),
            pl.BlockSpec((D, C), lambda i: (0, 0)),
            pl.BlockSpec((1, C), lambda i: (0, 0)),
        ],
        out_specs=[
            pl.BlockSpec((BN, D), lambda i: (i, 0)),
            pl.BlockSpec((BN, C), lambda i: (i, 0)),
        ],
        out_shape=[
            jax.ShapeDtypeStruct((N, D), _f32),
            jax.ShapeDtypeStruct((N, C), _f32),
        ],
    )(aggp, nd_col, W, b_row, Wc, bc_row)


@jax.jit
def kernel(g, x, W1, b1, W2, b2, Wc, bc):
    pad_nodes = N + jax.lax.rem(jnp.arange(EP - E, dtype=jnp.int32),
                                jnp.int32(NP - N))
    pad = jnp.stack([pad_nodes, pad_nodes])
    gidx = jnp.concatenate([g, pad], axis=1).reshape(2, NW * TPB, EB)

    norms = _degnorm(gidx)
    ns_col = norms[0].reshape(NP, 1)
    nd_col = norms[1].reshape(NP, 1)

    m1 = _scale_call(x, ns_col)
    aggp1 = _agg(gidx, m1)
    m2 = _mm1_call(aggp1, ns_col, nd_col, W1, b1.reshape(1, D))
    aggp2 = _agg(gidx, m2)
    h, logits = _mm2_call(aggp2, nd_col, W2, b2.reshape(1, D),
                          Wc, bc.reshape(1, C))
    return h, logits


# speedup vs baseline: 12.2465x; 1.0101x over previous
import functools
import jax
import jax.numpy as jnp
from jax import lax
from jax.experimental import pallas as pl
from jax.experimental.pallas import tpu as pltpu
from jax.experimental.pallas import tpu_sc as plsc

N = 10000
D = 128
C = 64
E = 320000

NC = 2
NS = 16
NW = NC * NS

NP = 10240
RPT = NP // NS
EB = 48
TPB = 216
KB = 4
EPT = TPB * EB
EP = NW * EPT
ERT = EP // NS // EB
DUMP = NP - 1

_mesh = plsc.VectorSubcoreMesh(core_axis_name="c", subcore_axis_name="s")
_f32 = jnp.float32


def _quake_rsqrt16(s):
    ii = plsc.bitcast(s, jnp.int32)
    ii = 0x5F3759DF - lax.shift_right_logical(ii, 1)
    y = plsc.bitcast(ii, _f32)
    hs = 0.5 * s
    for _ in range(3):
        y = y * (1.5 - hs * y * y)
    return jnp.where(s > 0.0, y, jnp.zeros((16,), _f32))


def _degnorm_body(gidx, norms_out, idx_v, deg_v, stage, acc_v, sh):
    cid = lax.axis_index("c")
    sid = lax.axis_index("s")

    pltpu.sync_copy(gidx.at[cid, pl.ds(sid * ERT, ERT)], idx_v)

    zeros16 = jnp.zeros((16,), _f32)

    def zb(i, carry):
        deg_v[pl.ds(i * 16, 16)] = zeros16
        return carry

    lax.fori_loop(0, NP // 16, zb, 0)

    ones16 = jnp.ones((16,), _f32)

    def sb(r, carry):
        for c in range(EB // 16):
            plsc.addupdate_scatter(deg_v, [idx_v[r, pl.ds(c * 16, 16)]],
                                   ones16)
        return carry

    lax.fori_loop(0, ERT, sb, 0)

    pltpu.sync_copy(deg_v, sh.at[sid])
    plsc.subcore_barrier()

    nb = sid * RPT
    for k in range(NS):
        pltpu.sync_copy(sh.at[k, pl.ds(nb, RPT)], stage.at[k])

    def rb(i, carry):
        s = stage[0, pl.ds(i * 16, 16)]
        for k in range(1, NS):
            s = s + stage[k, pl.ds(i * 16, 16)]
        acc_v[pl.ds(i * 16, 16)] = _quake_rsqrt16(s)
        return carry

    lax.fori_loop(0, RPT // 16, rb, 0)

    pltpu.sync_copy(acc_v, norms_out.at[cid, pl.ds(nb, RPT)])


@functools.partial(
    pl.kernel,
    out_type=jax.ShapeDtypeStruct((NC, NP), _f32),
    mesh=_mesh,
    scratch_types=[
        pltpu.VMEM((ERT, EB), jnp.int32),
        pltpu.VMEM((NP,), _f32),
        pltpu.VMEM((NS, RPT), _f32),
        pltpu.VMEM((RPT,), _f32),
        pltpu.VMEM_SHARED((NS, NP), _f32),
    ],
    compiler_params=pltpu.CompilerParams(needs_layout_passes=False, use_tc_tiling_on_sc=False),
)
def _degnorm(*args):
    _degnorm_body(*args)


def _make_agg():
    def body(gidx, m_hbm, out_hbm, srcv, dstv,
             rows0, rows1, rows2, rows3,
             agg_sh, gs0, gs1, gs2, gs3, ss0, ss1, ss2, ss3):
        cid = lax.axis_index("c")
        sid = lax.axis_index("s")
        wid = cid * NS + sid
        rows = (rows0, rows1, rows2, rows3)
        gsem = (gs0, gs1, gs2, gs3)
        ssem = (ss0, ss1, ss2, ss3)

        pltpu.sync_copy(gidx.at[0, pl.ds(wid * TPB, TPB)],
                        srcv.at[pl.ds(0, TPB)])
        pltpu.sync_copy(gidx.at[0, pl.ds(wid * TPB, 1)],
                        srcv.at[pl.ds(TPB, 1)])
        pltpu.sync_copy(gidx.at[0, pl.ds(wid * TPB, 1)],
                        srcv.at[pl.ds(TPB + 1, 1)])
        pltpu.sync_copy(gidx.at[1, pl.ds(wid * TPB, TPB)], dstv)

        zeros16 = jnp.zeros((16,), _f32)

        def zb(i, carry):
            for c in range(D // 16):
                for rb in rows:
                    rb[i, pl.ds(c * 16, 16)] = zeros16
            return carry

        lax.fori_loop(0, EB, zb, 0)

        for t in range(RPT // EB):
            pltpu.sync_copy(rows0, agg_sh.at[pl.ds(sid * RPT + t * EB, EB)])
        plsc.subcore_barrier()

        pltpu.async_copy(m_hbm.at[srcv.at[0]], rows0, gsem[0])
        pltpu.async_copy(m_hbm.at[srcv.at[1]], rows1, gsem[1])
        pltpu.async_copy(rows2, agg_sh.at[dstv.at[0]], ssem[2], add=True)
        pltpu.async_copy(rows3, agg_sh.at[dstv.at[0]], ssem[3], add=True)

        def slot(j, k, kn):
            pltpu.make_async_copy(rows[kn], agg_sh.at[dstv.at[0]],
                                  ssem[kn]).wait()
            pltpu.async_copy(m_hbm.at[srcv.at[j + 2]], rows[kn], gsem[kn])
            pltpu.make_async_copy(m_hbm.at[srcv.at[j]], rows[k],
                                  gsem[k]).wait()
            pltpu.async_copy(rows[k], agg_sh.at[dstv.at[j]], ssem[k],
                             add=True)

        def eb4(t, carry):
            j = 4 * t
            slot(j, 0, 2)
            slot(j + 1, 1, 3)
            slot(j + 2, 2, 0)
            slot(j + 3, 3, 1)
            return carry

        lax.fori_loop(0, TPB // KB, eb4, 0)
        pltpu.make_async_copy(rows[2], agg_sh.at[dstv.at[0]], ssem[2]).wait()
        pltpu.make_async_copy(rows[3], agg_sh.at[dstv.at[0]], ssem[3]).wait()
        pltpu.make_async_copy(m_hbm.at[srcv.at[0]], rows[0], gsem[0]).wait()
        pltpu.make_async_copy(m_hbm.at[srcv.at[1]], rows[1], gsem[1]).wait()
        plsc.subcore_barrier()

        pltpu.sync_copy(agg_sh.at[pl.ds(sid * RPT, RPT)],
                        out_hbm.at[cid, pl.ds(sid * RPT, RPT)])

    return pl.kernel(
        body,
        out_type=jax.ShapeDtypeStruct((NC, NP, D), _f32),
        mesh=_mesh,
        scratch_types=[
            pltpu.VMEM((TPB + 2, EB), jnp.int32),
            pltpu.VMEM((TPB, EB), jnp.int32),
            pltpu.VMEM((EB, D), _f32),
            pltpu.VMEM((EB, D), _f32),
            pltpu.VMEM((EB, D), _f32),
            pltpu.VMEM((EB, D), _f32),
            pltpu.VMEM_SHARED((NP, D), _f32),
            pltpu.SemaphoreType.DMA,
            pltpu.SemaphoreType.DMA,
            pltpu.SemaphoreType.DMA,
            pltpu.SemaphoreType.DMA,
            pltpu.SemaphoreType.DMA,
            pltpu.SemaphoreType.DMA,
            pltpu.SemaphoreType.DMA,
            pltpu.SemaphoreType.DMA,
        ],
        compiler_params=pltpu.CompilerParams(needs_layout_passes=False, use_tc_tiling_on_sc=False),
    )


_agg = _make_agg()


BN = 1024


def _scale_tc(x_ref, ns_ref, o_ref):
    o_ref[...] = x_ref[...] * ns_ref[...]


def _mm1_tc(agg_ref, ns_ref, nd_ref, w_ref, b_ref, o_ref):
    a = agg_ref[0] + agg_ref[1]
    p = jnp.dot(a, w_ref[...], preferred_element_type=_f32)
    o_ref[...] = jnp.maximum(nd_ref[...] * p + b_ref[...], 0.0) * ns_ref[...]


def _mm2_tc(agg_ref, nd_ref, w_ref, b_ref, wc_ref, bc_ref, h_ref, l_ref):
    a = agg_ref[0] + agg_ref[1]
    h = nd_ref[...] * jnp.dot(a, w_ref[...], preferred_element_type=_f32) \
        + b_ref[...]
    h_ref[...] = h
    l_ref[...] = jnp.dot(jnp.maximum(h, 0.0), wc_ref[...],
                         preferred_element_type=_f32) + bc_ref[...]


def _scale_call(x, ns_col):
    return pl.pallas_call(
        _scale_tc,
        grid=(NP // BN,),
        in_specs=[
            pl.BlockSpec((BN, D), lambda i: (i, 0)),
            pl.BlockSpec((BN, 1), lambda i: (i, 0)),
        ],
        out_specs=pl.BlockSpec((BN, D), lambda i: (i, 0)),
        out_shape=jax.ShapeDtypeStruct((NP, D), _f32),
    )(x, ns_col)


def _mm1_call(aggp, ns_col, nd_col, W, b_row):
    return pl.pallas_call(
        _mm1_tc,
        grid=(NP // BN,),
        in_specs=[
            pl.BlockSpec((NC, BN, D), lambda i: (0, i, 0)),
            pl.BlockSpec((BN, 1), lambda i: (i, 0)),
            pl.BlockSpec((BN, 1), lambda i: (i, 0)),
            pl.BlockSpec((D, D), lambda i: (0, 0)),
            pl.BlockSpec((1, D), lambda i: (0, 0)),
        ],
        out_specs=pl.BlockSpec((BN, D), lambda i: (i, 0)),
        out_shape=jax.ShapeDtypeStruct((NP, D), _f32),
    )(aggp, ns_col, nd_col, W, b_row)


def _mm2_call(aggp, nd_col, W, b_row, Wc, bc_row):
    return pl.pallas_call(
        _mm2_tc,
        grid=(NP // BN,),
        in_specs=[
            pl.BlockSpec((NC, BN, D), lambda i: (0, i, 0)),
            pl.BlockSpec((BN, 1), lambda i: (i, 0)),
            pl.BlockSpec((D, D), lambda i: (0, 0)),
            pl.BlockSpec((1, D), lambda i: (0, 0)),
            pl.BlockSpec((D, C), lambda i: (0, 0)),
            pl.BlockSpec((1, C), lambda i: (0, 0)),
        ],
        out_specs=[
            pl.BlockSpec((BN, D), lambda i: (i, 0)),
            pl.BlockSpec((BN, C), lambda i: (i, 0)),
        ],
        out_shape=[
            jax.ShapeDtypeStruct((N, D), _f32),
            jax.ShapeDtypeStruct((N, C), _f32),
        ],
    )(aggp, nd_col, W, b_row, Wc, bc_row)


@jax.jit
def kernel(g, x, W1, b1, W2, b2, Wc, bc):
    pad_nodes = N + jax.lax.rem(jnp.arange(EP - E, dtype=jnp.int32),
                                jnp.int32(NP - N))
    pad = jnp.stack([pad_nodes, pad_nodes])
    gidx = jnp.concatenate([g, pad], axis=1).reshape(2, NW * TPB, EB)

    norms = _degnorm(gidx)
    ns_col = norms[0].reshape(NP, 1)
    nd_col = norms[1].reshape(NP, 1)

    m1 = _scale_call(x, ns_col)
    aggp1 = _agg(gidx, m1)
    m2 = _mm1_call(aggp1, ns_col, nd_col, W1, b1.reshape(1, D))
    aggp2 = _agg(gidx, m2)
    h, logits = _mm2_call(aggp2, nd_col, W2, b2.reshape(1, D),
                          Wc, bc.reshape(1, C))
    return h, logits
